# baseline (speedup 1.0000x reference)
import functools

import jax
import jax.numpy as jnp
from jax.experimental import pallas as pl
from jax.experimental.pallas import tpu as pltpu


def _mlp(x0, w1_ref, b1_ref, gamma_ref, beta_ref, w2_ref, b2_ref):
    dn_t = (((1,), (1,)), ((), ()))
    x = jax.lax.dot_general(x0, w1_ref[...], dn_t,
                            preferred_element_type=jnp.float32) + b1_ref[...]
    mu = jnp.mean(x, axis=-1, keepdims=True)
    var = jnp.mean((x - mu) ** 2, axis=-1, keepdims=True)
    x = (x - mu) * jax.lax.rsqrt(var + 1e-5) * gamma_ref[...] + beta_ref[...]
    x = jnp.maximum(x, 0.0)
    return jax.lax.dot_general(x, w2_ref[...], dn_t,
                               preferred_element_type=jnp.float32) + b2_ref[...]


def _fused_body(batch_ref, h_ref, vn_h_ref, w1_ref, b1_ref, gamma_ref,
                beta_ref, w2_ref, b2_ref, out_ref, vn_out_ref,
                h_sc, sums_sc, counts_sc, vn_hi_sc, *, G, B, NB, GC):
    i = pl.program_id(0)
    nk = G // GC
    W = 2 * GC

    @pl.when(i < NB)
    def _():
        @pl.when(i == 0)
        def _():
            sums_sc[...] = jnp.zeros_like(sums_sc)
            counts_sc[...] = jnp.zeros_like(counts_sc)

        b = batch_ref[pl.ds(i, 1), :]
        g_lo = jnp.min(b)
        g_hi = jnp.max(b)
        g0 = (g_lo // GC) * GC
        hblk = h_ref[...].astype(jnp.bfloat16)
        h_sc[pl.ds(i * B, B), :] = hblk
        gids = jax.lax.broadcasted_iota(jnp.int32, (W, B), 0)
        dn = (((1,), (0,)), ((), ()))
        oh_t = (gids == jnp.broadcast_to(b - g0, (W, B)))
        oh_bf = oh_t.astype(jnp.bfloat16)
        part = jax.lax.dot_general(oh_bf, hblk, dn,
                                   preferred_element_type=jnp.float32)
        cnt = jnp.sum(oh_t.astype(jnp.float32), axis=1, keepdims=True)
        sums_sc[pl.ds(g0, W), :] += part
        counts_sc[pl.ds(g0, W), :] += cnt
        gids_t = jax.lax.broadcasted_iota(jnp.int32, (GC, B), 0)
        for k in range(2, nk):
            @pl.when((g_hi >= k * GC) & (k * GC >= g0 + W))
            def _(k=k):
                oh_tk = (gids_t == jnp.broadcast_to(b - (k * GC), (GC, B)))
                oh_bk = oh_tk.astype(jnp.bfloat16)
                pk = jax.lax.dot_general(oh_bk, hblk, dn,
                                         preferred_element_type=jnp.float32)
                ck = jnp.sum(oh_tk.astype(jnp.float32), axis=1, keepdims=True)
                sums_sc[pl.ds(k * GC, GC), :] += pk
                counts_sc[pl.ds(k * GC, GC), :] += ck

    @pl.when(i == NB)
    def _():
        mean = (sums_sc[pl.ds(0, G), :]
                / jnp.maximum(counts_sc[pl.ds(0, G), :], 1.0))
        vn_out = _mlp(mean + vn_h_ref[...], w1_ref, b1_ref, gamma_ref,
                      beta_ref, w2_ref, b2_ref)
        vn_out_ref[...] = vn_out
        vn_hi_sc[pl.ds(0, G), :] = vn_out.astype(jnp.bfloat16)
        vn_hi_sc[pl.ds(G, GC), :] = jnp.zeros((GC, vn_out.shape[1]),
                                              jnp.bfloat16)

    @pl.when(i > NB)
    def _():
        j = i - NB - 1
        b = batch_ref[pl.ds(j, 1), :]
        g_lo = jnp.min(b)
        g_hi = jnp.max(b)
        g0 = (g_lo // GC) * GC
        gids = jax.lax.broadcasted_iota(jnp.int32, (W, B), 0)
        dn = (((0,), (0,)), ((), ()))
        oh_bf = (gids == jnp.broadcast_to(b - g0, (W, B))).astype(jnp.bfloat16)
        g = jax.lax.dot_general(oh_bf, vn_hi_sc[pl.ds(g0, W), :], dn,
                                preferred_element_type=jnp.float32)
        out_ref[...] = h_sc[pl.ds(j * B, B), :].astype(jnp.float32) + g
        gids_t = jax.lax.broadcasted_iota(jnp.int32, (GC, B), 0)
        for k in range(2, nk):
            @pl.when((g_hi >= k * GC) & (k * GC >= g0 + W))
            def _(k=k):
                oh_bk = (gids_t == jnp.broadcast_to(b - (k * GC), (GC, B))
                         ).astype(jnp.bfloat16)
                gk = jax.lax.dot_general(
                    oh_bk, vn_hi_sc[pl.ds(k * GC, GC), :], dn,
                    preferred_element_type=jnp.float32)
                out_ref[...] += gk


def _pick_block(n):
    for cand in range(5000, 7, -8):
        if n % cand == 0:
            return cand
    return n


def kernel(h, batch, vn_h, W1, b1, gamma, beta, W2, b2, layer_idx):
    del layer_idx
    N, D = h.shape
    G = vn_h.shape[0]
    B = _pick_block(N)
    NB = N // B
    batch2 = batch.astype(jnp.int32).reshape(NB, B)

    h_out, vn_out = pl.pallas_call(
        functools.partial(_fused_body, G=G, B=B, NB=NB, GC=128),
        grid=(2 * NB + 1,),
        in_specs=[
            pl.BlockSpec((NB, B), lambda i: (0, 0)),
            pl.BlockSpec((B, D), lambda i: (jnp.minimum(i, NB - 1), 0)),
            pl.BlockSpec((G, D), lambda i: (0, 0)),
            pl.BlockSpec((D, D), lambda i: (0, 0)),
            pl.BlockSpec((1, D), lambda i: (0, 0)),
            pl.BlockSpec((1, D), lambda i: (0, 0)),
            pl.BlockSpec((1, D), lambda i: (0, 0)),
            pl.BlockSpec((D, D), lambda i: (0, 0)),
            pl.BlockSpec((1, D), lambda i: (0, 0)),
        ],
        out_specs=[
            pl.BlockSpec((B, D), lambda i: (jnp.maximum(i - NB - 1, 0), 0)),
            pl.BlockSpec((G, D), lambda i: (0, 0)),
        ],
        out_shape=[
            jax.ShapeDtypeStruct((N, D), jnp.float32),
            jax.ShapeDtypeStruct((G, D), jnp.float32),
        ],
        scratch_shapes=[
            pltpu.VMEM((N, D), jnp.bfloat16),
            pltpu.VMEM((G + 128, D), jnp.float32),
            pltpu.VMEM((G + 128, 1), jnp.float32),
            pltpu.VMEM((G + 128, D), jnp.bfloat16),
        ],
    )(batch2, h, vn_h, W1, b1.reshape(1, D), gamma.reshape(1, D),
      beta.reshape(1, D), W2, b2.reshape(1, D))

    return (h_out, vn_out)
